# Initial kernel scaffold; baseline (speedup 1.0000x reference)
import jax
import jax.numpy as jnp
from jax.experimental import pallas as pl


def kernel(from_up, from_down, gemm_faces, W_up, b_up, W_c1, b_c1, W_c2, b_c2):
    raise NotImplementedError("write your pallas kernel here")


# speedup vs baseline: 11.4326x; 11.4326x over previous
import functools

import jax
import jax.numpy as jnp
from jax import lax
from jax.experimental import pallas as pl
from jax.experimental.pallas import tpu as pltpu
from jax.experimental.pallas import tpu_sc as plsc

C = 128
TF = 512
NW = 32
CB = 112


def _pad_to(f):
    m = 512 * 7
    return ((f + m - 1) // m) * m


def _tc1_body(xu, xd, wu, wb, bu, ys, ya, yb, yc, yd):
    y = lax.dot_general(xu[...], wu[...], (((0,), (0,)), ((), ())),
                        preferred_element_type=jnp.float32)
    ys[...] = y[:, 0:C] + bu[...]
    ya[...] = y[:, C:2 * C]
    yb[...] = y[:, 2 * C:3 * C]
    yc[...] = y[:, 3 * C:4 * C]
    yd[...] = lax.dot_general(xd[...], wb[...], (((0,), (0,)), ((), ())),
                              preferred_element_type=jnp.float32)


def _tc2_body(ys, ga, gb, gc, yd, wa, bc, zs, za, zb, zc):
    x1 = ys[...] + ga[...] + gb[...] + gc[...]
    z = jnp.dot(x1, wa[...], preferred_element_type=jnp.float32) + yd[...]
    zs[...] = z[:, 0:C] + bc[...]
    za[...] = z[:, C:2 * C]
    zb[...] = z[:, 2 * C:3 * C]
    zc[...] = z[:, 3 * C:4 * C]


def _tc3_body(zs, ha, hb, hc, x2):
    x2[...] = jnp.maximum(zs[...] + ha[...] + hb[...] + hc[...], 0.0)


def _tc4_body(x2, ka, kb, kc, w2, b2, out):
    x4 = jnp.concatenate([x2[...], ka[...], kb[...], kc[...]], axis=1)
    o = lax.dot_general(w2[...], x4, (((0,), (1,)), ((), ())),
                        preferred_element_type=jnp.float32)
    out[...] = jnp.maximum(o + b2[...], 0.0)


def _row_spec(bf):
    return pl.BlockSpec((bf, C), lambda i: (i, 0))


def _full_spec(shape):
    return pl.BlockSpec(shape, lambda i: (0,) * len(shape))


def _sc_gather3_body(n_chunks, ta, tb, tc, i0, i1, i2, ga, gb, gc,
                     x0, x1, x2, ba, bb, bc, sa, sb, sc_):
    rows_per_w = n_chunks * CB
    wid = lax.axis_index("s") * 2 + lax.axis_index("c")
    for ci in range(n_chunks):
        base = wid * rows_per_w + ci * CB
        pltpu.sync_copy(i0.at[pl.ds(base, CB)], x0)
        pltpu.sync_copy(i1.at[pl.ds(base, CB)], x1)
        pltpu.sync_copy(i2.at[pl.ds(base, CB)], x2)
        ca = pltpu.async_copy(ta.at[x0], ba, sa)
        cb = pltpu.async_copy(tb.at[x1], bb, sb)
        cc = pltpu.async_copy(tc.at[x2], bc, sc_)
        ca.wait()
        cb.wait()
        cc.wait()
        pltpu.sync_copy(ba, ga.at[pl.ds(base, CB)])
        pltpu.sync_copy(bb, gb.at[pl.ds(base, CB)])
        pltpu.sync_copy(bc, gc.at[pl.ds(base, CB)])


def _sc_gather3(ta, tb, tc, i0, i1, i2):
    fp = ta.shape[0]
    n_chunks = fp // (NW * CB)
    mesh = plsc.VectorSubcoreMesh(core_axis_name="c", subcore_axis_name="s")
    row = jax.ShapeDtypeStruct((fp, C), jnp.float32)
    return pl.kernel(
        functools.partial(_sc_gather3_body, n_chunks),
        out_type=(row, row, row),
        mesh=mesh,
        scratch_types=[
            pltpu.VMEM((CB,), jnp.int32),
            pltpu.VMEM((CB,), jnp.int32),
            pltpu.VMEM((CB,), jnp.int32),
            pltpu.VMEM((CB, C), jnp.float32),
            pltpu.VMEM((CB, C), jnp.float32),
            pltpu.VMEM((CB, C), jnp.float32),
            pltpu.SemaphoreType.DMA,
            pltpu.SemaphoreType.DMA,
            pltpu.SemaphoreType.DMA,
        ],
    )(ta, tb, tc, i0, i1, i2)


def kernel(from_up, from_down, gemm_faces, W_up, b_up, W_c1, b_c1, W_c2, b_c2):
    f = from_up.shape[2]
    fp = _pad_to(f)
    grid = fp // TF

    xu = from_up[0]
    xd = from_down[0]

    wu = W_up[:, :, 0, :].transpose(1, 2, 0).reshape(C, 4 * C)
    wa = W_c1[:, :C, 0, :].transpose(1, 2, 0).reshape(C, 4 * C)
    wb = W_c1[:, C:, 0, :].transpose(1, 2, 0).reshape(C, 4 * C)
    w2 = W_c2[:, :, 0, :].transpose(2, 1, 0).reshape(4 * C, C)
    w2 = w2 + jnp.concatenate([jnp.eye(C, dtype=jnp.float32),
                               jnp.zeros((3 * C, C), jnp.float32)], axis=0)
    bu = b_up.reshape(1, C)
    bc1 = b_c1.reshape(1, C)
    b2 = b_c2.reshape(C, 1)

    idx = jnp.pad(gemm_faces[0], ((0, fp - f), (0, 0)))
    n0, n1, n2 = idx[:, 0], idx[:, 1], idx[:, 2]

    row = jax.ShapeDtypeStruct((fp, C), jnp.float32)
    wide = jax.ShapeDtypeStruct((fp, 4 * C), jnp.float32)

    ys, ya, yb, yc, yd = pl.pallas_call(
        _tc1_body,
        grid=(grid,),
        in_specs=[
            pl.BlockSpec((C, TF), lambda i: (0, i)),
            pl.BlockSpec((C, TF), lambda i: (0, i)),
            _full_spec((C, 4 * C)),
            _full_spec((C, 4 * C)),
            _full_spec((1, C)),
        ],
        out_specs=[_row_spec(TF)] * 4 + [pl.BlockSpec((TF, 4 * C),
                                                      lambda i: (i, 0))],
        out_shape=[row, row, row, row, wide],
    )(xu, xd, wu, wb, bu)

    ga, gb, gc = _sc_gather3(ya, yb, yc, n0, n1, n2)

    zs, za, zb, zc = pl.pallas_call(
        _tc2_body,
        grid=(grid,),
        in_specs=[_row_spec(TF)] * 4 + [
            pl.BlockSpec((TF, 4 * C), lambda i: (i, 0)),
            _full_spec((C, 4 * C)),
            _full_spec((1, C)),
        ],
        out_specs=[_row_spec(TF)] * 4,
        out_shape=[row, row, row, row],
    )(ys, ga, gb, gc, yd, wa, bc1)

    ha, hb, hc = _sc_gather3(za, zb, zc, n0, n1, n2)

    x2 = pl.pallas_call(
        _tc3_body,
        grid=(grid,),
        in_specs=[_row_spec(TF)] * 4,
        out_specs=_row_spec(TF),
        out_shape=row,
    )(zs, ha, hb, hc)

    ka, kb, kc = _sc_gather3(x2, x2, x2, n0, n1, n2)

    out = pl.pallas_call(
        _tc4_body,
        grid=(grid,),
        in_specs=[_row_spec(TF)] * 4 + [
            _full_spec((4 * C, C)),
            _full_spec((C, 1)),
        ],
        out_specs=pl.BlockSpec((C, TF), lambda i: (0, i)),
        out_shape=jax.ShapeDtypeStruct((C, f), jnp.float32),
    )(x2, ka, kb, kc, w2, b2)

    return out[None]


# speedup vs baseline: 12.2300x; 1.0697x over previous
import functools

import jax
import jax.numpy as jnp
from jax import lax
from jax.experimental import pallas as pl
from jax.experimental.pallas import tpu as pltpu
from jax.experimental.pallas import tpu_sc as plsc

C = 128
TF = 512
NW = 32
CB = 112
NBUF = 3


def _pad_to(f):
    m = 512 * 7
    return ((f + m - 1) // m) * m


def _tc1_body(xu, xd, wu, wb, bu, ys, yt, yd):
    y = lax.dot_general(xu[...], wu[...], (((0,), (0,)), ((), ())),
                        preferred_element_type=jnp.float32)
    ys[...] = y[:, 0:C] + bu[...]
    yt[0] = y[:, C:2 * C]
    yt[1] = y[:, 2 * C:3 * C]
    yt[2] = y[:, 3 * C:4 * C]
    yd[...] = lax.dot_general(xd[...], wb[...], (((0,), (0,)), ((), ())),
                              preferred_element_type=jnp.float32)


def _tc2_body(ys, g, yd, wa, bc, zs, zt):
    x1 = ys[...] + g[0] + g[1] + g[2]
    z = jnp.dot(x1, wa[...], preferred_element_type=jnp.float32) + yd[...]
    zs[...] = z[:, 0:C] + bc[...]
    zt[0] = z[:, C:2 * C]
    zt[1] = z[:, 2 * C:3 * C]
    zt[2] = z[:, 3 * C:4 * C]


def _tc3_body(zs, h, x2):
    x2[...] = jnp.maximum(zs[...] + h[0] + h[1] + h[2], 0.0)


def _tc4_body(x2, k, w2, b2, out):
    x4 = jnp.concatenate([x2[...], k[0], k[1], k[2]], axis=1)
    o = lax.dot_general(w2[...], x4, (((0,), (1,)), ((), ())),
                        preferred_element_type=jnp.float32)
    out[...] = jnp.maximum(o + b2[...], 0.0)


def _row_spec(bf):
    return pl.BlockSpec((bf, C), lambda i: (i, 0))


def _tap_spec():
    return pl.BlockSpec((3, TF, C), lambda i: (0, i, 0))


def _full_spec(shape):
    return pl.BlockSpec(shape, lambda i: (0,) * len(shape))


def _sc_gather_body(n_chunks, table, idxh, out, idx_v, bufs, gsems, wsems):
    rows_per_w = n_chunks * CB
    wid = lax.axis_index("s") * 2 + lax.axis_index("c")
    pltpu.sync_copy(idxh.at[wid], idx_v)
    gathers = [None] * NBUF
    writes = [None] * NBUF
    for ci in range(min(NBUF, n_chunks)):
        gathers[ci] = pltpu.async_copy(table.at[idx_v.at[ci]], bufs[ci],
                                       gsems[ci])
    for ci in range(n_chunks):
        b = ci % NBUF
        gathers[b].wait()
        writes[b] = pltpu.async_copy(
            bufs[b], out.at[pl.ds(wid * rows_per_w + ci * CB, CB)], wsems[b])
        nc = ci + NBUF
        if nc < n_chunks:
            writes[b].wait()
            gathers[b] = pltpu.async_copy(table.at[idx_v.at[nc]], bufs[b],
                                          gsems[b])
    for b in range(min(NBUF, n_chunks)):
        if writes[b] is not None:
            writes[b].wait()


def _sc_gather(table3, idxh):
    n_chunks = idxh.shape[1]
    mesh = plsc.VectorSubcoreMesh(core_axis_name="c", subcore_axis_name="s")
    return pl.kernel(
        functools.partial(_sc_gather_body, n_chunks),
        out_type=jax.ShapeDtypeStruct((NW * n_chunks * CB, C), jnp.float32),
        mesh=mesh,
        scratch_types=[
            pltpu.VMEM((n_chunks, CB), jnp.int32),
            [pltpu.VMEM((CB, C), jnp.float32) for _ in range(NBUF)],
            [pltpu.SemaphoreType.DMA for _ in range(NBUF)],
            [pltpu.SemaphoreType.DMA for _ in range(NBUF)],
        ],
    )(table3, idxh)


def kernel(from_up, from_down, gemm_faces, W_up, b_up, W_c1, b_c1, W_c2, b_c2):
    f = from_up.shape[2]
    fp = _pad_to(f)
    grid = fp // TF
    n_chunks = 3 * fp // (NW * CB)

    xu = from_up[0]
    xd = from_down[0]

    wu = W_up[:, :, 0, :].transpose(1, 2, 0).reshape(C, 4 * C)
    wa = W_c1[:, :C, 0, :].transpose(1, 2, 0).reshape(C, 4 * C)
    wb = W_c1[:, C:, 0, :].transpose(1, 2, 0).reshape(C, 4 * C)
    w2 = W_c2[:, :, 0, :].transpose(2, 1, 0).reshape(4 * C, C)
    w2 = w2 + jnp.concatenate([jnp.eye(C, dtype=jnp.float32),
                               jnp.zeros((3 * C, C), jnp.float32)], axis=0)
    bu = b_up.reshape(1, C)
    bc1 = b_c1.reshape(1, C)
    b2 = b_c2.reshape(C, 1)

    nbr = jnp.pad(gemm_faces[0], ((0, fp - f), (0, 0))).T
    offs = (jnp.arange(3, dtype=jnp.int32) * fp)[:, None]
    idx_t = (nbr + offs).reshape(NW, n_chunks, CB)
    idx_p = nbr.reshape(NW, n_chunks, CB)

    row = jax.ShapeDtypeStruct((fp, C), jnp.float32)
    tap = jax.ShapeDtypeStruct((3, fp, C), jnp.float32)
    wide = jax.ShapeDtypeStruct((fp, 4 * C), jnp.float32)

    ys, yt, yd = pl.pallas_call(
        _tc1_body,
        grid=(grid,),
        in_specs=[
            pl.BlockSpec((C, TF), lambda i: (0, i)),
            pl.BlockSpec((C, TF), lambda i: (0, i)),
            _full_spec((C, 4 * C)),
            _full_spec((C, 4 * C)),
            _full_spec((1, C)),
        ],
        out_specs=[_row_spec(TF), _tap_spec(),
                   pl.BlockSpec((TF, 4 * C), lambda i: (i, 0))],
        out_shape=[row, tap, wide],
    )(xu, xd, wu, wb, bu)

    g = _sc_gather(yt.reshape(3 * fp, C), idx_t).reshape(3, fp, C)

    zs, zt = pl.pallas_call(
        _tc2_body,
        grid=(grid,),
        in_specs=[_row_spec(TF), _tap_spec(),
                  pl.BlockSpec((TF, 4 * C), lambda i: (i, 0)),
                  _full_spec((C, 4 * C)), _full_spec((1, C))],
        out_specs=[_row_spec(TF), _tap_spec()],
        out_shape=[row, tap],
    )(ys, g, yd, wa, bc1)

    h = _sc_gather(zt.reshape(3 * fp, C), idx_t).reshape(3, fp, C)

    x2 = pl.pallas_call(
        _tc3_body,
        grid=(grid,),
        in_specs=[_row_spec(TF), _tap_spec()],
        out_specs=_row_spec(TF),
        out_shape=row,
    )(zs, h)

    k = _sc_gather(x2, idx_p).reshape(3, fp, C)

    out = pl.pallas_call(
        _tc4_body,
        grid=(grid,),
        in_specs=[_row_spec(TF), _tap_spec(),
                  _full_spec((4 * C, C)), _full_spec((C, 1))],
        out_specs=pl.BlockSpec((C, TF), lambda i: (0, i)),
        out_shape=jax.ShapeDtypeStruct((C, f), jnp.float32),
    )(x2, k, w2, b2)

    return out[None]


# speedup vs baseline: 14.1009x; 1.1530x over previous
import functools

import jax
import jax.numpy as jnp
from jax import lax
from jax.experimental import pallas as pl
from jax.experimental.pallas import tpu as pltpu
from jax.experimental.pallas import tpu_sc as plsc

C = 128
TF = 512
NW = 32
CB = 112
NBUF = 3


def _pad_to(f):
    m = 512 * 7
    return ((f + m - 1) // m) * m


def _tc1_body(xu, wu, bu, ys, yt):
    y = jnp.dot(xu[...], wu[...], preferred_element_type=jnp.float32)
    ys[...] = y[:, 0:C] + bu[...]
    yt[0] = y[:, C:2 * C]
    yt[1] = y[:, 2 * C:3 * C]
    yt[2] = y[:, 3 * C:4 * C]


def _tc2_body(ys, g, xd, wa, wb, bc, zs, zt):
    x1 = ys[...] + g[0] + g[1] + g[2]
    z = (jnp.dot(x1, wa[...], preferred_element_type=jnp.float32)
         + jnp.dot(xd[...], wb[...], preferred_element_type=jnp.float32))
    zs[...] = z[:, 0:C] + bc[...]
    zt[0] = z[:, C:2 * C]
    zt[1] = z[:, 2 * C:3 * C]
    zt[2] = z[:, 3 * C:4 * C]


def _tc3_body(zs, h, x2):
    x2[...] = jnp.maximum(zs[...] + h[0] + h[1] + h[2], 0.0)


def _tc4_body(x2, k, w2, b2, out):
    x4 = jnp.concatenate([x2[...], k[0], k[1], k[2]], axis=1)
    o = jnp.dot(x4, w2[...], preferred_element_type=jnp.float32)
    out[...] = jnp.maximum(o + b2[...], 0.0)


def _row_spec(bf):
    return pl.BlockSpec((bf, C), lambda i: (i, 0))


def _tap_spec():
    return pl.BlockSpec((3, TF, C), lambda i: (0, i, 0))


def _full_spec(shape):
    return pl.BlockSpec(shape, lambda i: (0,) * len(shape))


def _sc_gather_body(n_chunks, table, idxh, out, idx_v, bufs, gsems, wsems):
    rows_per_w = n_chunks * CB
    wid = lax.axis_index("s") * 2 + lax.axis_index("c")
    pltpu.sync_copy(idxh.at[wid], idx_v)
    gathers = [None] * NBUF
    writes = [None] * NBUF
    for ci in range(min(NBUF, n_chunks)):
        gathers[ci] = pltpu.async_copy(table.at[idx_v.at[ci]], bufs[ci],
                                       gsems[ci])
    for ci in range(n_chunks):
        b = ci % NBUF
        gathers[b].wait()
        writes[b] = pltpu.async_copy(
            bufs[b], out.at[pl.ds(wid * rows_per_w + ci * CB, CB)], wsems[b])
        nc = ci + NBUF
        if nc < n_chunks:
            writes[b].wait()
            gathers[b] = pltpu.async_copy(table.at[idx_v.at[nc]], bufs[b],
                                          gsems[b])
    for b in range(min(NBUF, n_chunks)):
        if writes[b] is not None:
            writes[b].wait()


def _sc_gather(table3, idxh):
    n_chunks = idxh.shape[1]
    mesh = plsc.VectorSubcoreMesh(core_axis_name="c", subcore_axis_name="s")
    return pl.kernel(
        functools.partial(_sc_gather_body, n_chunks),
        out_type=jax.ShapeDtypeStruct((NW * n_chunks * CB, C), jnp.float32),
        mesh=mesh,
        scratch_types=[
            pltpu.VMEM((n_chunks, CB), jnp.int32),
            [pltpu.VMEM((CB, C), jnp.float32) for _ in range(NBUF)],
            [pltpu.SemaphoreType.DMA for _ in range(NBUF)],
            [pltpu.SemaphoreType.DMA for _ in range(NBUF)],
        ],
    )(table3, idxh)


def kernel(from_up, from_down, gemm_faces, W_up, b_up, W_c1, b_c1, W_c2, b_c2):
    f = from_up.shape[2]
    fp = _pad_to(f)
    grid = fp // TF
    n_chunks = 3 * fp // (NW * CB)

    xu = from_up[0].T
    xd = from_down[0].T

    wu = W_up[:, :, 0, :].transpose(1, 2, 0).reshape(C, 4 * C)
    wa = W_c1[:, :C, 0, :].transpose(1, 2, 0).reshape(C, 4 * C)
    wb = W_c1[:, C:, 0, :].transpose(1, 2, 0).reshape(C, 4 * C)
    w2 = W_c2[:, :, 0, :].transpose(2, 1, 0).reshape(4 * C, C)
    w2 = w2 + jnp.concatenate([jnp.eye(C, dtype=jnp.float32),
                               jnp.zeros((3 * C, C), jnp.float32)], axis=0)
    bu = b_up.reshape(1, C)
    bc1 = b_c1.reshape(1, C)
    b2 = b_c2.reshape(1, C)

    nbr = jnp.pad(gemm_faces[0], ((0, fp - f), (0, 0))).T
    offs = (jnp.arange(3, dtype=jnp.int32) * fp)[:, None]
    idx_t = (nbr + offs).reshape(NW, n_chunks, CB)
    idx_p = nbr.reshape(NW, n_chunks, CB)

    row = jax.ShapeDtypeStruct((fp, C), jnp.float32)
    tap = jax.ShapeDtypeStruct((3, fp, C), jnp.float32)

    ys, yt = pl.pallas_call(
        _tc1_body,
        grid=(grid,),
        in_specs=[_row_spec(TF), _full_spec((C, 4 * C)), _full_spec((1, C))],
        out_specs=[_row_spec(TF), _tap_spec()],
        out_shape=[row, tap],
    )(xu, wu, bu)

    g = _sc_gather(yt.reshape(3 * fp, C), idx_t).reshape(3, fp, C)

    zs, zt = pl.pallas_call(
        _tc2_body,
        grid=(grid,),
        in_specs=[_row_spec(TF), _tap_spec(), _row_spec(TF),
                  _full_spec((C, 4 * C)), _full_spec((C, 4 * C)),
                  _full_spec((1, C))],
        out_specs=[_row_spec(TF), _tap_spec()],
        out_shape=[row, tap],
    )(ys, g, xd, wa, wb, bc1)

    h = _sc_gather(zt.reshape(3 * fp, C), idx_t).reshape(3, fp, C)

    x2 = pl.pallas_call(
        _tc3_body,
        grid=(grid,),
        in_specs=[_row_spec(TF), _tap_spec()],
        out_specs=_row_spec(TF),
        out_shape=row,
    )(zs, h)

    k = _sc_gather(x2, idx_p).reshape(3, fp, C)

    out = pl.pallas_call(
        _tc4_body,
        grid=(grid,),
        in_specs=[_row_spec(TF), _tap_spec(),
                  _full_spec((4 * C, C)), _full_spec((1, C))],
        out_specs=_row_spec(TF),
        out_shape=jax.ShapeDtypeStruct((f, C), jnp.float32),
    )(x2, k, w2, b2)

    return out.T[None]


# speedup vs baseline: 18.2340x; 1.2931x over previous
import functools

import jax
import jax.numpy as jnp
from jax import lax
from jax.experimental import pallas as pl
from jax.experimental.pallas import tpu as pltpu
from jax.experimental.pallas import tpu_sc as plsc

C = 128
TF = 512
NW = 32
CB = 64
NV = C // 16


def _pad_to(f):
    m = NW * CB
    return ((f + m - 1) // m) * m


def _tc1_body(xu, wu, bup, ys, yt):
    y = jnp.dot(xu[...], wu[...], preferred_element_type=jnp.float32)
    y = y + bup[...]
    ys[...] = y[:, 0:C]
    yt[0] = y[:, C:2 * C]
    yt[1] = y[:, 2 * C:3 * C]
    yt[2] = y[:, 3 * C:4 * C]


def _tc2_body(x1, xd, wa, wb, bcp, zs, zt):
    z = (jnp.dot(x1[...], wa[...], preferred_element_type=jnp.float32)
         + jnp.dot(xd[...], wb[...], preferred_element_type=jnp.float32))
    z = z + bcp[...]
    zs[...] = z[:, 0:C]
    zt[0] = z[:, C:2 * C]
    zt[1] = z[:, 2 * C:3 * C]
    zt[2] = z[:, 3 * C:4 * C]


def _tc3_body(x2, w2, b2p, vs, vt):
    v = jnp.dot(x2[...], w2[...], preferred_element_type=jnp.float32)
    v = v + b2p[...]
    vs[...] = v[:, 0:C]
    vt[0] = v[:, C:2 * C]
    vt[1] = v[:, 2 * C:3 * C]
    vt[2] = v[:, 3 * C:4 * C]


def _row_spec(bf):
    return pl.BlockSpec((bf, C), lambda i: (i, 0))


def _tap_spec():
    return pl.BlockSpec((3, TF, C), lambda i: (0, i, 0))


def _full_spec(shape):
    return pl.BlockSpec(shape, lambda i: (0,) * len(shape))


def _sc_acc_body(n_chunks, relu, tap, self_t, idxh, out,
                 idx_v, bt, bs, bo, gsems, ssems, wsems):
    rows_per_w = n_chunks * CB
    wid = lax.axis_index("s") * 2 + lax.axis_index("c")
    pltpu.sync_copy(idxh.at[wid], idx_v)

    def issue(ci, s):
        ds = [pltpu.async_copy(tap.at[idx_v.at[ci, k]], bt[s].at[k], gsems[s])
              for k in range(3)]
        base = wid * rows_per_w + ci * CB
        ds.append(pltpu.async_copy(self_t.at[pl.ds(base, CB)], bs[s], ssems[s]))
        return ds

    def accumulate(s):
        bt_s, bs_s, bo_s = bt[s], bs[s], bo[s]

        def body(r, carry):
            for u in range(NV):
                sl = pl.ds(u * 16, 16)
                acc = bs_s[r, sl] + bt_s[0, r, sl]
                acc = acc + bt_s[1, r, sl]
                acc = acc + bt_s[2, r, sl]
                if relu:
                    acc = jnp.maximum(acc, 0.0)
                bo_s[r, sl] = acc
            return carry

        lax.fori_loop(0, CB, body, 0)

    pend = [None, None]
    wr = [None, None]
    pend[0] = issue(0, 0)
    for ci in range(n_chunks):
        s = ci % 2
        if ci + 1 < n_chunks:
            pend[1 - s] = issue(ci + 1, 1 - s)
        for d in pend[s]:
            d.wait()
        if wr[s] is not None:
            wr[s].wait()
        accumulate(s)
        base = wid * rows_per_w + ci * CB
        wr[s] = pltpu.async_copy(bo[s], out.at[pl.ds(base, CB)], wsems[s])
    for s in (0, 1):
        if wr[s] is not None:
            wr[s].wait()


def _sc_acc(tap3, self_t, idxh, relu):
    n_chunks = idxh.shape[1]
    fp = self_t.shape[0]
    mesh = plsc.VectorSubcoreMesh(core_axis_name="c", subcore_axis_name="s")
    return pl.kernel(
        functools.partial(_sc_acc_body, n_chunks, relu),
        out_type=jax.ShapeDtypeStruct((fp, C), jnp.float32),
        mesh=mesh,
        scratch_types=[
            pltpu.VMEM((n_chunks, 3, CB), jnp.int32),
            [pltpu.VMEM((3, CB, C), jnp.float32) for _ in range(2)],
            [pltpu.VMEM((CB, C), jnp.float32) for _ in range(2)],
            [pltpu.VMEM((CB, C), jnp.float32) for _ in range(2)],
            [pltpu.SemaphoreType.DMA for _ in range(2)],
            [pltpu.SemaphoreType.DMA for _ in range(2)],
            [pltpu.SemaphoreType.DMA for _ in range(2)],
        ],
    )(tap3, self_t, idxh)


def kernel(from_up, from_down, gemm_faces, W_up, b_up, W_c1, b_c1, W_c2, b_c2):
    f = from_up.shape[2]
    fp = _pad_to(f)
    grid = fp // TF
    n_chunks = fp // (NW * CB)

    xu = from_up[0].T
    xd = from_down[0].T

    wu = W_up[:, :, 0, :].transpose(1, 2, 0).reshape(C, 4 * C)
    wa = W_c1[:, :C, 0, :].transpose(1, 2, 0).reshape(C, 4 * C)
    wb = W_c1[:, C:, 0, :].transpose(1, 2, 0).reshape(C, 4 * C)
    w2 = W_c2[:, :, 0, :].transpose(1, 2, 0).reshape(C, 4 * C)
    w2 = w2.at[:, :C].add(jnp.eye(C, dtype=jnp.float32))
    zpad = jnp.zeros((3 * C,), jnp.float32)
    bup = jnp.concatenate([b_up, zpad]).reshape(1, 4 * C)
    bcp = jnp.concatenate([b_c1, zpad]).reshape(1, 4 * C)
    b2p = jnp.concatenate([b_c2, zpad]).reshape(1, 4 * C)

    nbr = jnp.pad(gemm_faces[0], ((0, fp - f), (0, 0))).T
    offs = (jnp.arange(3, dtype=jnp.int32) * fp)[:, None]
    idx = (nbr + offs).reshape(3, NW, n_chunks, CB).transpose(1, 2, 0, 3)

    row = jax.ShapeDtypeStruct((fp, C), jnp.float32)
    tap = jax.ShapeDtypeStruct((3, fp, C), jnp.float32)

    ys, yt = pl.pallas_call(
        _tc1_body,
        grid=(grid,),
        in_specs=[_row_spec(TF), _full_spec((C, 4 * C)),
                  _full_spec((1, 4 * C))],
        out_specs=[_row_spec(TF), _tap_spec()],
        out_shape=[row, tap],
    )(xu, wu, bup)
    x1 = _sc_acc(yt.reshape(3 * fp, C), ys, idx, relu=False)

    zs, zt = pl.pallas_call(
        _tc2_body,
        grid=(grid,),
        in_specs=[_row_spec(TF), _row_spec(TF),
                  _full_spec((C, 4 * C)), _full_spec((C, 4 * C)),
                  _full_spec((1, 4 * C))],
        out_specs=[_row_spec(TF), _tap_spec()],
        out_shape=[row, tap],
    )(x1, xd, wa, wb, bcp)
    x2 = _sc_acc(zt.reshape(3 * fp, C), zs, idx, relu=True)

    vs, vt = pl.pallas_call(
        _tc3_body,
        grid=(grid,),
        in_specs=[_row_spec(TF), _full_spec((C, 4 * C)),
                  _full_spec((1, 4 * C))],
        out_specs=[_row_spec(TF), _tap_spec()],
        out_shape=[row, tap],
    )(x2, w2, b2p)
    out = _sc_acc(vt.reshape(3 * fp, C), vs, idx, relu=True)

    return out[:f].T[None]
